# baseline (speedup 1.0000x reference)
import functools

import jax
import jax.numpy as jnp
from jax import lax
from jax.experimental import pallas as pl
from jax.experimental.pallas import tpu as pltpu
from jax.experimental.pallas import tpu_sc as plsc

IDXW = 128
NBUF = 3
LANES = 16
DPAD = 128


def _build(B, L, V, D, NC, NS):
    NW = NC * NS
    seqs_w = B // NW
    n_chunks = seqs_w
    rem = L - IDXW

    mesh = plsc.VectorSubcoreMesh(
        core_axis_name="c", subcore_axis_name="s",
        num_cores=NC, num_subcores=NS)

    @functools.partial(
        pl.kernel,
        out_type=jax.ShapeDtypeStruct((B, L, D), jnp.float32),
        mesh=mesh,
        scratch_types=[
            pltpu.VMEM((seqs_w, L), jnp.int32),
            pltpu.VMEM((L, D), jnp.float32),
            pltpu.VMEM((NBUF, L, DPAD), jnp.float32),
            pltpu.SemaphoreType.DMA,
            pltpu.SemaphoreType.DMA,
            pltpu.SemaphoreType.DMA,
            pltpu.SemaphoreType.DMA,
            pltpu.SemaphoreType.DMA,
            pltpu.SemaphoreType.DMA,
        ],
        compiler_params=pltpu.CompilerParams(use_tc_tiling_on_sc=False),
    )
    def body(idx_hbm, table_hbm, pos_hbm, out_hbm,
             idx_v, pos_v, rows_v, g0, g1, g2, o0, o1, o2):
        gsems = (g0, g1, g2)
        osems = (o0, o1, o2)
        wid = lax.axis_index("s") * NC + lax.axis_index("c")
        seq_base = wid * seqs_w

        pltpu.sync_copy(pos_hbm, pos_v)
        pltpu.sync_copy(idx_hbm.at[pl.ds(seq_base, seqs_w)], idx_v)

        def fire_gather(c, b):
            pltpu.async_copy(
                table_hbm.at[idx_v.at[c, pl.ds(0, IDXW)]],
                rows_v.at[b, pl.ds(0, IDXW)],
                gsems[b])
            pltpu.async_copy(
                table_hbm.at[idx_v.at[c, pl.ds(IDXW, rem)]],
                rows_v.at[b, pl.ds(IDXW, rem)],
                gsems[b])

        def drain_gather(b):
            pltpu.make_async_copy(
                table_hbm.at[pl.ds(0, L)], rows_v.at[b], gsems[b]).wait()

        def drain_store(b):
            pltpu.make_async_copy(
                rows_v.at[b, pl.ds(0, L), pl.ds(0, D)],
                out_hbm.at[0], osems[b]).wait()

        def trip(c, b):
            drain_gather(b)

            @pl.loop(0, L, unroll=4)
            def _add(r):
                for q in range(D // LANES):
                    plsc.addupdate(
                        rows_v.at[b, r, pl.ds(q * LANES, LANES)],
                        pos_v[r, pl.ds(q * LANES, LANES)])

            pltpu.async_copy(
                rows_v.at[b, pl.ds(0, L), pl.ds(0, D)],
                out_hbm.at[seq_base + c],
                osems[b])
            bn = (b + 2) % NBUF

            @pl.when(jnp.logical_and(c >= 1, c + 2 < n_chunks))
            def _():
                drain_store(bn)

            @pl.when(c + 2 < n_chunks)
            def _():
                fire_gather(c + 2, bn)

        fire_gather(0, 0)
        fire_gather(1, 1)

        n_main = (n_chunks // NBUF) * NBUF

        @pl.loop(0, n_main, step=NBUF)
        def _outer(t):
            for db in range(NBUF):
                trip(t + db, db)

        for c in range(n_main, n_chunks):
            trip(c, c % NBUF)

        for c in range(n_chunks - NBUF, n_chunks):
            drain_store(c % NBUF)

    return body


def kernel(inputs, token_table, pos_table):
    B, L = inputs.shape
    V, D = token_table.shape
    info = plsc.get_sparse_core_info()
    NC, NS = info.num_cores, info.num_subcores
    tbl128 = jnp.pad(token_table, ((0, 0), (0, DPAD - D)))
    out = _build(B, L, V, D, NC, NS)(
        inputs.astype(jnp.int32), tbl128, pos_table)
    return out
